# baseline (speedup 1.0000x reference)
import jax
import jax.numpy as jnp
from jax import lax
from jax.experimental import pallas as pl
from jax.experimental.pallas import tpu as pltpu
from jax.experimental.pallas import tpu_sc as plsc

N_SPARSE = 26
N_DENSE = 13
VOCAB = 1000000
BATCH = 16384

NC = 2
NS = 16
NW = NC * NS
BPW = BATCH // NW
CHUNKS = BPW // 16
ROWW = 16
NBUF = 6


def _sc_body(idx_hbm, xd_hbm, w_hbm, tables_hbm, out_hbm, *scratch):
    idx_bufs = scratch[:N_SPARSE]
    row_bufs = scratch[N_SPARSE:2 * N_SPARSE]
    gbufs = scratch[2 * N_SPARSE:2 * N_SPARSE + NBUF]
    sems = scratch[2 * N_SPARSE + NBUF:2 * N_SPARSE + 2 * NBUF]
    xd_v, w_v, out_v, sem = scratch[2 * N_SPARSE + 2 * NBUF:]

    wid = lax.axis_index("s") * NC + lax.axis_index("c")

    stage = [pltpu.async_copy(idx_hbm.at[wid, f], idx_bufs[f], sem)
             for f in range(N_SPARSE)]
    stage.append(pltpu.async_copy(xd_hbm.at[wid], xd_v, sem))
    stage.append(pltpu.async_copy(w_hbm, w_v, sem))
    for h in stage:
        h.wait()

    def rows_body(c, _):
        o = c * 16
        for f in range(N_SPARSE):
            row_bufs[f][pl.ds(o, 16)] = lax.shift_right_logical(
                idx_bufs[f][pl.ds(o, 16)], 4)
        return _

    lax.fori_loop(0, CHUNKS, rows_body, None)

    def fire(f):
        return pltpu.async_copy(
            tables_hbm.at[f].at[row_bufs[f]], gbufs[f % NBUF], sems[f % NBUF])

    handles = [fire(f) for f in range(NBUF)]

    wb = [w_v[j, :] for j in range(N_DENSE)]

    def dense_body(c, _):
        o = c * 16
        acc = xd_v[0, pl.ds(o, 16)] * wb[0]
        for j in range(1, N_DENSE):
            acc = acc + xd_v[j, pl.ds(o, 16)] * wb[j]
        out_v[pl.ds(o, 16)] = acc
        return _

    lax.fori_loop(0, CHUNKS, dense_body, None)

    lanes = lax.iota(jnp.int32, 16)
    for f in range(N_SPARSE):
        handles[f % NBUF].wait()
        gbuf = gbufs[f % NBUF]

        def acc_body(c, _, f=f, gbuf=gbuf):
            o = c * 16
            col = idx_bufs[f][pl.ds(o, 16)] & 15
            v = plsc.load_gather(gbuf, [o + lanes, col])
            out_v[pl.ds(o, 16)] = out_v[pl.ds(o, 16)] + v
            return _

        lax.fori_loop(0, CHUNKS, acc_body, None)
        if f + NBUF < N_SPARSE:
            handles[f % NBUF] = fire(f + NBUF)

    pltpu.sync_copy(out_v, out_hbm.at[wid])


@jax.jit
def kernel(X, tables, dense_weight):
    idx = X[:, :N_SPARSE].astype(jnp.int32)
    idx_blk = idx.T.reshape(N_SPARSE, NW, BPW).swapaxes(0, 1)
    xd_blk = X[:, N_SPARSE:].T.reshape(N_DENSE, NW, BPW).swapaxes(0, 1)
    w_rep = jnp.broadcast_to(dense_weight.reshape(N_DENSE, 1), (N_DENSE, 16))
    tables3d = tables.reshape(N_SPARSE, VOCAB // ROWW, ROWW)

    mesh = plsc.VectorSubcoreMesh(core_axis_name="c", subcore_axis_name="s")
    scratch = (
        [pltpu.VMEM((BPW,), jnp.int32) for _ in range(N_SPARSE)]
        + [pltpu.VMEM((BPW,), jnp.int32) for _ in range(N_SPARSE)]
        + [pltpu.VMEM((BPW, ROWW), jnp.float32) for _ in range(NBUF)]
        + [pltpu.SemaphoreType.DMA for _ in range(NBUF)]
        + [
            pltpu.VMEM((N_DENSE, BPW), jnp.float32),
            pltpu.VMEM((N_DENSE, 16), jnp.float32),
            pltpu.VMEM((BPW,), jnp.float32),
            pltpu.SemaphoreType.DMA,
        ]
    )
    out = pl.kernel(
        _sc_body,
        out_type=jax.ShapeDtypeStruct((NW, BPW), jnp.float32),
        mesh=mesh,
        scratch_types=scratch,
        compiler_params=pltpu.CompilerParams(needs_layout_passes=False, use_tc_tiling_on_sc=False),
    )(idx_blk, xd_blk, w_rep, tables3d)
    return out.reshape(BATCH, 1)
